# Initial kernel scaffold; baseline (speedup 1.0000x reference)
import functools

import jax
import jax.numpy as jnp
from jax import lax
from jax.experimental import pallas as pl
from jax.experimental.pallas import tpu as pltpu
from jax.experimental.pallas import tpu_sc as plsc

NC, NS, L = 2, 16, 16
NW = NC * NS
B, S, D = 4, 2048, 1024
PPT = S // NW
CH = 32
NCH = PPT // CH
NV = D // L
STEPS = [(c, b) for c in range(NCH) for b in range(B)]
NIT = len(STEPS)

_mesh = plsc.VectorSubcoreMesh(
    core_axis_name="c", subcore_axis_name="s", num_cores=NC, num_subcores=NS
)


@functools.partial(
    pl.kernel,
    out_type=jax.ShapeDtypeStruct((B, S, D), jnp.float32),
    mesh=_mesh,
    scratch_types=[
        pltpu.VMEM((B, PPT), jnp.int32),
        pltpu.VMEM((CH, D), jnp.float32),
        pltpu.VMEM((2, CH, D), jnp.float32),
        pltpu.SemaphoreType.DMA,
        pltpu.SemaphoreType.DMA,
        pltpu.SemaphoreType.DMA,
        pltpu.SemaphoreType.DMA,
    ],
)
def _emb_kernel(x_hbm, emb_hbm, pos_hbm, out_hbm, idx_v, pos_v, tok_v,
                g0, g1, o0, o1):
    wid = lax.axis_index("s") * NC + lax.axis_index("c")
    pbase = wid * PPT
    gs = [g0, g1]
    osems = [o0, o1]
    gdesc = [None, None]
    odesc = [None, None]

    pltpu.sync_copy(x_hbm.at[:, pl.ds(pbase, PPT)], idx_v)

    def start_gather(s):
        c, b = STEPS[s]
        buf = s % 2
        gdesc[buf] = pltpu.async_copy(
            emb_hbm.at[idx_v.at[b, pl.ds(c * CH, CH)]],
            tok_v.at[buf],
            gs[buf],
        )

    def add_pos(buf):
        def row_body(r, carry):
            for j in range(NV):
                sl = pl.ds(j * L, L)
                plsc.addupdate(tok_v.at[buf, r, sl], pos_v[r, sl])
            return carry
        lax.fori_loop(0, CH, row_body, 0)

    start_gather(0)
    for s in range(NIT):
        c, b = STEPS[s]
        buf = s % 2
        if b == 0:
            pltpu.sync_copy(pos_hbm.at[pl.ds(pbase + c * CH, CH)], pos_v)
        if s + 1 < NIT:
            nbuf = (s + 1) % 2
            if odesc[nbuf] is not None:
                odesc[nbuf].wait()
            start_gather(s + 1)
        gdesc[buf].wait()
        add_pos(buf)
        odesc[buf] = pltpu.async_copy(
            tok_v.at[buf],
            out_hbm.at[b, pl.ds(pbase + c * CH, CH)],
            osems[buf],
        )
    odesc[0].wait()
    odesc[1].wait()


def kernel(x, emb, pos_emb):
    return _emb_kernel(jnp.asarray(x, jnp.int32), emb, pos_emb)


# speedup vs baseline: 1.2935x; 1.2935x over previous
import functools

import jax
import jax.numpy as jnp
from jax import lax
from jax.experimental import pallas as pl
from jax.experimental.pallas import tpu as pltpu
from jax.experimental.pallas import tpu_sc as plsc

NC, NS, L = 2, 16, 16
NW = NC * NS
B, S, D = 4, 2048, 1024
PPT = S // NW
CH = 32
NCH = PPT // CH
NV = D // L
STEPS = [(c, b) for c in range(NCH) for b in range(B)]
NIT = len(STEPS)

_mesh = plsc.VectorSubcoreMesh(
    core_axis_name="c", subcore_axis_name="s", num_cores=NC, num_subcores=NS
)


@functools.partial(
    pl.kernel,
    out_type=jax.ShapeDtypeStruct((B, S, D), jnp.float32),
    mesh=_mesh,
    scratch_types=[
        pltpu.VMEM((B, S), jnp.int32),
        pltpu.VMEM((CH, D), jnp.float32),
        pltpu.VMEM((2, CH, D), jnp.float32),
        pltpu.SemaphoreType.DMA,
        pltpu.SemaphoreType.DMA,
        pltpu.SemaphoreType.DMA,
        pltpu.SemaphoreType.DMA,
    ],
)
def _emb_kernel(x_hbm, emb_hbm, pos_hbm, out_hbm, idx_v, pos_v, tok_v,
                g0, g1, o0, o1):
    wid = lax.axis_index("s") * NC + lax.axis_index("c")
    pbase = wid * PPT
    gs = [g0, g1]
    osems = [o0, o1]
    gdesc = [None, None]
    odesc = [None, None]

    pltpu.sync_copy(x_hbm, idx_v)

    def start_gather(s):
        c, b = STEPS[s]
        buf = s % 2
        gdesc[buf] = pltpu.async_copy(
            emb_hbm.at[idx_v.at[b, pl.ds(pbase + c * CH, CH)]],
            tok_v.at[buf],
            gs[buf],
        )

    def add_pos(buf):
        def row_body(r, carry):
            for j in range(NV):
                sl = pl.ds(j * L, L)
                plsc.addupdate(tok_v.at[buf, r, sl], pos_v[r, sl])
            return carry
        lax.fori_loop(0, CH, row_body, 0)

    start_gather(0)
    for s in range(NIT):
        c, b = STEPS[s]
        buf = s % 2
        if b == 0:
            pltpu.sync_copy(pos_hbm.at[pl.ds(pbase + c * CH, CH)], pos_v)
        if s + 1 < NIT:
            nbuf = (s + 1) % 2
            if odesc[nbuf] is not None:
                odesc[nbuf].wait()
            start_gather(s + 1)
        gdesc[buf].wait()
        add_pos(buf)
        odesc[buf] = pltpu.async_copy(
            tok_v.at[buf],
            out_hbm.at[b, pl.ds(pbase + c * CH, CH)],
            osems[buf],
        )
    odesc[0].wait()
    odesc[1].wait()


def kernel(x, emb, pos_emb):
    return _emb_kernel(jnp.asarray(x, jnp.int32), emb, pos_emb)
